# baseline (speedup 1.0000x reference)
import functools

import jax
import jax.numpy as jnp
from jax import lax
from jax.experimental import pallas as pl
from jax.experimental.pallas import tpu as pltpu
from jax.experimental.pallas import tpu_sc as plsc

T, B, C = 576, 64, 768
KEEP = 144
N_OUT = KEEP * B
L = 16

NC, NS = 2, 16
NW = NC * NS
ROWS_PER_W = N_OUT // NW
CHUNK = 32
NCHUNK = ROWS_PER_W // CHUNK
NBUF = 4


BWD_W = 24
BWD_ROWS_PER_W = T // BWD_W


@functools.partial(
    pl.kernel,
    out_type=(
        jax.ShapeDtypeStruct((N_OUT, C), jnp.float32),
        jax.ShapeDtypeStruct((T, B), jnp.int32),
    ),
    mesh=plsc.VectorSubcoreMesh(core_axis_name="c", subcore_axis_name="s"),
    scratch_types=(
        pltpu.VMEM((ROWS_PER_W,), jnp.int32),
        pltpu.VMEM((ROWS_PER_W,), jnp.int32),
        pltpu.VMEM((BWD_ROWS_PER_W, B), jnp.int32),
        [pltpu.VMEM((CHUNK, C), jnp.float32) for _ in range(NBUF)],
        [pltpu.SemaphoreType.DMA for _ in range(NBUF)],
        [pltpu.SemaphoreType.DMA for _ in range(NBUF)],
    ),
)
def _gather_rows(fwd_hbm, table_hbm, bwd_hbm, out_hbm, bwd_out_hbm,
                 raw_v, idx_v, bwd_v, bufs, gsems, ssems):
    wid = lax.axis_index("s") * NC + lax.axis_index("c")
    base = wid * ROWS_PER_W

    pltpu.sync_copy(fwd_hbm.at[pl.ds(base, ROWS_PER_W)], raw_v)

    lane = lax.iota(jnp.int32, L)
    for k in range(ROWS_PER_W // L):
        jbase = lax.rem(base + k * L, B)
        idx_v[pl.ds(k * L, L)] = raw_v[pl.ds(k * L, L)] * B + jbase + lane

    gathers = [None] * NCHUNK
    stores = [None] * NCHUNK

    def start_store(d):
        bd = bufs[d % NBUF]
        gathers[d].wait()
        stores[d] = pltpu.async_copy(
            bd, out_hbm.at[pl.ds(base + d * CHUNK, CHUNK)], ssems[d % NBUF])

    for c in range(NCHUNK):
        if c >= NBUF:
            stores[c - NBUF].wait()
        gathers[c] = pltpu.async_copy(
            table_hbm.at[idx_v.at[pl.ds(c * CHUNK, CHUNK)]],
            bufs[c % NBUF], gsems[c % NBUF])
        if c == 0:
            @pl.when(wid < BWD_W)
            def _copy_bwd():
                brow = wid * BWD_ROWS_PER_W
                pltpu.sync_copy(
                    bwd_hbm.at[pl.ds(brow, BWD_ROWS_PER_W)], bwd_v)
                pltpu.sync_copy(
                    bwd_v, bwd_out_hbm.at[pl.ds(brow, BWD_ROWS_PER_W)])
        if c >= NBUF - 1:
            start_store(c - (NBUF - 1))
    for d in range(NCHUNK - NBUF + 1, NCHUNK):
        start_store(d)
    for d in range(NCHUNK - NBUF, NCHUNK):
        stores[d].wait()


def kernel(patches, forward_indexes, backward_indexes):
    table = patches.reshape(T * B, C)
    fwd_flat = forward_indexes.reshape(-1)
    out, bwd_out = _gather_rows(fwd_flat, table, backward_indexes)
    return out.reshape(KEEP, B, C), bwd_out


# speedup vs baseline: 1.0398x; 1.0398x over previous
import functools

import jax
import jax.numpy as jnp
from jax import lax
from jax.experimental import pallas as pl
from jax.experimental.pallas import tpu as pltpu
from jax.experimental.pallas import tpu_sc as plsc

T, B, C = 576, 64, 768
KEEP = 144
N_OUT = KEEP * B
L = 16

NC, NS = 2, 16
NW = NC * NS
ROWS_PER_W = N_OUT // NW
CHUNK = 16
NCHUNK = ROWS_PER_W // CHUNK
NBUF = 8


@functools.partial(
    pl.kernel,
    out_type=jax.ShapeDtypeStruct((N_OUT, C), jnp.float32),
    mesh=plsc.VectorSubcoreMesh(core_axis_name="c", subcore_axis_name="s"),
    scratch_types=(
        pltpu.VMEM((ROWS_PER_W,), jnp.int32),
        pltpu.VMEM((ROWS_PER_W,), jnp.int32),
        [pltpu.VMEM((CHUNK, C), jnp.float32) for _ in range(NBUF)],
        [pltpu.SemaphoreType.DMA for _ in range(NBUF)],
        [pltpu.SemaphoreType.DMA for _ in range(NBUF)],
    ),
)
def _gather_rows(fwd_hbm, table_hbm, out_hbm, raw_v, idx_v, bufs, gsems, ssems):
    wid = lax.axis_index("s") * NC + lax.axis_index("c")
    base = wid * ROWS_PER_W

    pltpu.sync_copy(fwd_hbm.at[pl.ds(base, ROWS_PER_W)], raw_v)

    lane = lax.iota(jnp.int32, L)
    for k in range(ROWS_PER_W // L):
        jbase = lax.rem(base + k * L, B)
        idx_v[pl.ds(k * L, L)] = raw_v[pl.ds(k * L, L)] * B + jbase + lane

    gathers = [None] * NCHUNK
    stores = [None] * NCHUNK

    def start_store(d):
        bd = bufs[d % NBUF]
        gathers[d].wait()
        stores[d] = pltpu.async_copy(
            bd, out_hbm.at[pl.ds(base + d * CHUNK, CHUNK)], ssems[d % NBUF])

    for c in range(NCHUNK):
        if c >= NBUF:
            stores[c - NBUF].wait()
        gathers[c] = pltpu.async_copy(
            table_hbm.at[idx_v.at[pl.ds(c * CHUNK, CHUNK)]],
            bufs[c % NBUF], gsems[c % NBUF])
        if c >= NBUF - 1:
            start_store(c - (NBUF - 1))
    for d in range(NCHUNK - NBUF + 1, NCHUNK):
        start_store(d)
    for d in range(NCHUNK - NBUF, NCHUNK):
        stores[d].wait()


def kernel(patches, forward_indexes, backward_indexes):
    table = patches.reshape(T * B, C)
    fwd_flat = forward_indexes.reshape(-1)
    out = _gather_rows(fwd_flat, table)
    return out.reshape(KEEP, B, C), backward_indexes


# speedup vs baseline: 1.0401x; 1.0002x over previous
import functools

import jax
import jax.numpy as jnp
from jax import lax
from jax.experimental import pallas as pl
from jax.experimental.pallas import tpu as pltpu
from jax.experimental.pallas import tpu_sc as plsc

T, B, C = 576, 64, 768
KEEP = 144
N_OUT = KEEP * B
L = 16

NC, NS = 2, 16
NW = NC * NS
ROWS_PER_W = N_OUT // NW
CHUNK = 16
NCHUNK = ROWS_PER_W // CHUNK
NBUF = 8


@functools.partial(
    pl.kernel,
    out_type=jax.ShapeDtypeStruct((N_OUT, C), jnp.float32),
    mesh=plsc.VectorSubcoreMesh(core_axis_name="c", subcore_axis_name="s"),
    scratch_types=(
        pltpu.VMEM((ROWS_PER_W,), jnp.int32),
        pltpu.VMEM((ROWS_PER_W,), jnp.int32),
        [pltpu.VMEM((CHUNK, C), jnp.float32) for _ in range(NBUF)],
        [pltpu.SemaphoreType.DMA for _ in range(NBUF)],
        [pltpu.SemaphoreType.DMA for _ in range(NBUF)],
    ),
)
def _gather_rows(fwd_hbm, table_hbm, out_hbm, raw_v, idx_v, bufs, gsems, ssems):
    wid = lax.axis_index("s") * NC + lax.axis_index("c")
    base = wid * ROWS_PER_W

    pltpu.sync_copy(fwd_hbm.at[pl.ds(base, ROWS_PER_W)], raw_v)

    lane = lax.iota(jnp.int32, L)
    for k in range(ROWS_PER_W // L):
        jbase = lax.rem(base + k * L, B)
        idx_v[pl.ds(k * L, L)] = raw_v[pl.ds(k * L, L)] * B + jbase + lane

    gathers = [None] * NCHUNK
    stores = [None] * NCHUNK

    def start_store(d):
        bd = bufs[d % NBUF]
        gathers[d].wait()
        stores[d] = pltpu.async_copy(
            bd, out_hbm.at[pl.ds(base + d * CHUNK, CHUNK)], ssems[d % NBUF])

    for c in range(NCHUNK):
        if c >= NBUF:
            stores[c - NBUF].wait()
        gathers[c] = pltpu.async_copy(
            table_hbm.at[idx_v.at[pl.ds(c * CHUNK, CHUNK)]],
            bufs[c % NBUF], gsems[c % NBUF])
        if c >= NBUF - 1:
            start_store(c - (NBUF - 1))
    for d in range(NCHUNK - NBUF + 1, NCHUNK):
        start_store(d)
    for d in range(NCHUNK - NBUF, NCHUNK):
        stores[d].wait()


def kernel(patches, forward_indexes, backward_indexes):
    table = patches.reshape(T * B, C)
    fwd_flat = forward_indexes[:KEEP].reshape(-1)
    out = _gather_rows(fwd_flat, table)
    return out.reshape(KEEP, B, C), backward_indexes
